# baseline (speedup 1.0000x reference)
import functools

import jax
import jax.numpy as jnp
from jax import lax
from jax.experimental import pallas as pl
from jax.experimental.pallas import tpu as pltpu
from jax.experimental.pallas import tpu_sc as plsc

N = 50000
E = 800000
F = 128
H = 16

NC = 2
NS = 16
NW = NC * NS
TPW = 128
KT = 10
WIN = KT * TPW
NWIN = 21
EPT = WIN * NWIN
EPAD = EPT * NW
NPT = N // NS

RB = 5000
GRID = N // RB


def _gelu(v):
    return 0.5 * v * (1.0 + lax.erf(v * (2.0 ** -0.5)))


def _prep_layer(p):
    s = p["gamma"] / jnp.sqrt(p["var"] + 1e-3)
    t = p["beta"] - p["mean"] * s
    return [s.reshape(1, -1), t.reshape(1, -1), p["W"], p["b"].reshape(1, -1)]


def _prep_ffn(layers):
    out = []
    for p in layers:
        out.extend(_prep_layer(p))
    return out


def _apply_ffn_refs(v, refs):
    for i in range(0, len(refs), 4):
        s, t, W, b = refs[i:i + 4]
        v = _gelu(jnp.dot(v * s[...] + t[...], W[...],
                          preferred_element_type=jnp.float32) + b[...])
    return v


def _full_spec(a):
    nd = a.ndim
    return pl.BlockSpec(a.shape, lambda i: (0,) * nd)


def _row_spec(cols):
    return pl.BlockSpec((RB, cols), lambda i: (i, 0))


def _stage_a(node, pre_flat, msg_flat):
    n_pre, n_msg = len(pre_flat), len(msg_flat)

    def body(node_ref, *refs):
        pre_refs = refs[:n_pre]
        msg_refs = refs[n_pre:n_pre + n_msg]
        x_out, y_out = refs[n_pre + n_msg:]
        v = _apply_ffn_refs(node_ref[...], pre_refs)
        x_out[...] = v
        y_out[...] = _apply_ffn_refs(v, msg_refs)

    flat = list(pre_flat) + list(msg_flat)
    return pl.pallas_call(
        body,
        grid=(GRID,),
        in_specs=[_row_spec(F)] + [_full_spec(a) for a in flat],
        out_specs=[_row_spec(H), _row_spec(H)],
        out_shape=[jax.ShapeDtypeStruct((N, H), jnp.float32)] * 2,
        compiler_params=pltpu.CompilerParams(
            dimension_semantics=("arbitrary",)),
    )(node, *flat)


def _stage_b(x, agg_a, agg_b, upd_split, msg_flat):
    n_upd, n_msg = len(upd_split), len(msg_flat)

    def body(x_ref, aa_ref, ab_ref, *refs):
        upd_refs = refs[:n_upd]
        msg_refs = refs[n_upd:n_upd + n_msg]
        x2_out, y_out = refs[n_upd + n_msg:]
        x_v = x_ref[...]
        agg = aa_ref[...] + ab_ref[...]
        sx, tx, sa, ta, Wx, Wa, b1 = upd_refs[:7]
        u = _gelu(jnp.dot(x_v * sx[...] + tx[...], Wx[...],
                          preferred_element_type=jnp.float32)
                  + jnp.dot(agg * sa[...] + ta[...], Wa[...],
                            preferred_element_type=jnp.float32) + b1[...])
        u = _apply_ffn_refs(u, upd_refs[7:])
        nrm = jnp.maximum(jnp.sum(u * u, axis=-1, keepdims=True), 1e-12)
        x2 = u * lax.rsqrt(nrm) + x_v
        x2_out[...] = x2
        y_out[...] = _apply_ffn_refs(x2, msg_refs)

    flat = list(upd_split) + list(msg_flat)
    return pl.pallas_call(
        body,
        grid=(GRID,),
        in_specs=[_row_spec(H)] * 3 + [_full_spec(a) for a in flat],
        out_specs=[_row_spec(H), _row_spec(H)],
        out_shape=[jax.ShapeDtypeStruct((N, H), jnp.float32)] * 2,
        compiler_params=pltpu.CompilerParams(
            dimension_semantics=("arbitrary",)),
    )(x, agg_a, agg_b, *flat)


def _stage_c(x, agg_a, agg_b, upd_split, post_flat, wl, bl, wc, bc):
    n_upd, n_post = len(upd_split), len(post_flat)

    def body(x_ref, aa_ref, ab_ref, *refs):
        upd_refs = refs[:n_upd]
        post_refs = refs[n_upd:n_upd + n_post]
        wl_ref, bl_ref, wc_ref, bc_ref, out_ref = refs[n_upd + n_post:]
        i = pl.program_id(0)
        x_v = x_ref[...]
        agg = aa_ref[...] + ab_ref[...]
        sx, tx, sa, ta, Wx, Wa, b1 = upd_refs[:7]
        u = _gelu(jnp.dot(x_v * sx[...] + tx[...], Wx[...],
                          preferred_element_type=jnp.float32)
                  + jnp.dot(agg * sa[...] + ta[...], Wa[...],
                            preferred_element_type=jnp.float32) + b1[...])
        u = _apply_ffn_refs(u, upd_refs[7:])
        nrm = jnp.maximum(jnp.sum(u * u, axis=-1, keepdims=True), 1e-12)
        x3 = u * lax.rsqrt(nrm) + x_v
        v = _apply_ffn_refs(x3, post_refs)
        logit = jnp.dot(v, wl_ref[...],
                        preferred_element_type=jnp.float32) + bl_ref[...]

        @pl.when(i == 0)
        def _():
            out_ref[...] = bc_ref[...]

        out_ref[...] += jnp.sum(logit * wc_ref[...]).reshape(1, 1)

    flat = list(upd_split) + list(post_flat)
    return pl.pallas_call(
        body,
        grid=(GRID,),
        in_specs=([_row_spec(H)] * 3 + [_full_spec(a) for a in flat]
                  + [_full_spec(wl), _full_spec(bl),
                     pl.BlockSpec((RB, 1), lambda i: (i, 0)),
                     _full_spec(bc)]),
        out_specs=pl.BlockSpec((1, 1), lambda i: (0, 0)),
        out_shape=jax.ShapeDtypeStruct((1, 1), jnp.float32),
        compiler_params=pltpu.CompilerParams(
            dimension_semantics=("arbitrary",)),
    )(x, agg_a, agg_b, *flat, wl, bl, wc, bc)


def _spmm(y, nbr3, dst3, bw2, zrows):
    mesh = plsc.VectorSubcoreMesh(core_axis_name="c", subcore_axis_name="s")

    @functools.partial(
        pl.kernel,
        out_type=jax.ShapeDtypeStruct((NC, N, H), jnp.float32),
        mesh=mesh,
        scratch_types=[
            pltpu.VMEM((3, KT, TPW), jnp.int32),
            pltpu.VMEM((3, KT, TPW), jnp.int32),
            pltpu.VMEM((3, WIN), jnp.float32),
            pltpu.VMEM((3, WIN, H), jnp.float32),
            pltpu.VMEM_SHARED((N, H), jnp.float32),
            pltpu.SemaphoreType.DMA,
            pltpu.SemaphoreType.DMA,
            pltpu.SemaphoreType.DMA,
            pltpu.SemaphoreType.DMA,
            pltpu.SemaphoreType.DMA,
            pltpu.SemaphoreType.DMA,
        ],
        compiler_params=pltpu.CompilerParams(
            use_tc_tiling_on_sc=False, needs_layout_passes=False),
    )
    def k(y_hbm, nbr_hbm, dst_hbm, bw_hbm, z_hbm, out_hbm,
          gidx_v, sidx_v, bw_v, rows_v, agg_sh,
          gsem0, gsem1, gsem2, ssem0, ssem1, ssem2):
        c = lax.axis_index("c")
        s = lax.axis_index("s")
        wid = s * NC + c
        gsem = (gsem0, gsem1, gsem2)
        ssem = (ssem0, ssem1, ssem2)
        pltpu.sync_copy(z_hbm, agg_sh.at[pl.ds(s * NPT, NPT), :])
        plsc.subcore_barrier()

        def load_idx(w, p):
            pltpu.sync_copy(nbr_hbm.at[wid, pl.ds(w * KT, KT), :],
                            gidx_v.at[p])
            pltpu.sync_copy(dst_hbm.at[wid, pl.ds(w * KT, KT), :],
                            sidx_v.at[p])
            pltpu.sync_copy(bw_hbm.at[wid, pl.ds(w * WIN, WIN)], bw_v.at[p])

        def fire_gathers(p):
            for j in range(KT):
                pltpu.async_copy(y_hbm.at[gidx_v.at[p, j]],
                                 rows_v.at[p, pl.ds(j * TPW, TPW), :],
                                 gsem[p])

        def drain_g(p):
            pltpu.make_async_copy(y_hbm.at[pl.ds(0, WIN), :],
                                  rows_v.at[p], gsem[p]).wait()

        def multiply(p):
            def mul(g, cc):
                for l in range(16):
                    e = g * 16 + l
                    splat = plsc.load_gather(
                        bw_v.at[p], [jnp.full((16,), e, jnp.int32)])
                    rows_v[p, e, :] = rows_v[p, e, :] * splat
                return cc

            lax.fori_loop(0, WIN // 16, mul, 0)

        def fire_scatters(p):
            for j in range(KT):
                pltpu.async_copy(rows_v.at[p, pl.ds(j * TPW, TPW), :],
                                 agg_sh.at[sidx_v.at[p, j]], ssem[p],
                                 add=True)

        def drain_s(p):
            pltpu.make_async_copy(y_hbm.at[pl.ds(0, WIN), :],
                                  rows_v.at[p], ssem[p]).wait()

        def step(w, p, pn, drain_next, prefetch):
            drain_g(p)
            multiply(p)
            fire_scatters(p)
            if prefetch:
                if drain_next:
                    drain_s(pn)
                load_idx(w + 1, pn)
                fire_gathers(pn)

        load_idx(0, 0)
        fire_gathers(0)
        step(jnp.int32(0), 0, 1, False, True)
        step(jnp.int32(1), 1, 2, False, True)
        step(jnp.int32(2), 2, 0, True, True)

        def group(i, cc):
            base = 3 + 3 * i
            step(base, 0, 1, True, True)
            step(base + 1, 1, 2, True, True)
            step(base + 2, 2, 0, True, True)
            return cc

        lax.fori_loop(0, (NWIN - 6) // 3, group, 0)

        base = jnp.int32(NWIN - 3)
        step(base, 0, 1, True, True)
        step(base + 1, 1, 2, True, True)
        step(base + 2, 2, 0, False, False)
        drain_s(0)
        drain_s(1)
        drain_s(2)
        plsc.subcore_barrier()
        pltpu.sync_copy(agg_sh.at[pl.ds(s * NPT, NPT), :],
                        out_hbm.at[c, pl.ds(s * NPT, NPT), :])

    return k(y, nbr3, dst3, bw2, zrows)


def _prep_upd_split(layers):
    p0 = layers[0]
    s = p0["gamma"] / jnp.sqrt(p0["var"] + 1e-3)
    t = p0["beta"] - p0["mean"] * s
    out = [s[:H].reshape(1, H), t[:H].reshape(1, H),
           s[H:].reshape(1, H), t[H:].reshape(1, H),
           p0["W"][:H], p0["W"][H:], p0["b"].reshape(1, -1)]
    for p in layers[1:]:
        out.extend(_prep_layer(p))
    return out


def kernel(node_info, branches, branch_info, params):
    node = node_info[0]
    nbr = branches[1].astype(jnp.int32)
    dst = branches[0].astype(jnp.int32)
    bw = branch_info[0, :, 0]

    pad = EPAD - E
    pidx = lax.iota(jnp.int32, pad) % N
    nbr3 = jnp.concatenate([nbr, pidx]).reshape(NW, EPT // TPW, TPW)
    dst3 = jnp.concatenate([dst, pidx]).reshape(NW, EPT // TPW, TPW)
    bw2 = jnp.concatenate([bw, jnp.zeros((pad,), jnp.float32)]).reshape(NW, EPT)
    zrows = jnp.zeros((NPT, H), jnp.float32)

    pre_flat = _prep_ffn(params["pre"])
    msg1_flat = _prep_ffn(params["conv1"]["msg"])
    msg2_flat = _prep_ffn(params["conv2"]["msg"])
    upd1_split = _prep_upd_split(params["conv1"]["upd"])
    upd2_split = _prep_upd_split(params["conv2"]["upd"])
    post_flat = _prep_ffn(params["post"])
    wl = params["logits"]["W"]
    bl = params["logits"]["b"].reshape(1, 1)
    wc = params["critic"]["W"]
    bc = params["critic"]["b"].reshape(1, 1)

    x, y1 = _stage_a(node, pre_flat, msg1_flat)
    agg1 = _spmm(y1, nbr3, dst3, bw2, zrows)
    x2, y2 = _stage_b(x, agg1[0], agg1[1], upd1_split, msg2_flat)
    agg2 = _spmm(y2, nbr3, dst3, bw2, zrows)
    return _stage_c(x2, agg2[0], agg2[1], upd2_split, post_flat, wl, bl, wc, bc)


# speedup vs baseline: 1.0611x; 1.0611x over previous
import functools

import jax
import jax.numpy as jnp
from jax import lax
from jax.experimental import pallas as pl
from jax.experimental.pallas import tpu as pltpu
from jax.experimental.pallas import tpu_sc as plsc

N = 50000
E = 800000
F = 128
H = 16

NC = 2
NS = 16
NW = NC * NS
TPW = 128
KT = 20
WIN = KT * TPW
NWIN = 10
EPT = WIN * NWIN
EPAD = EPT * NW
NPT = N // NS

RB = 5000
GRID = N // RB


def _gelu(v):
    return 0.5 * v * (1.0 + lax.erf(v * (2.0 ** -0.5)))


def _prep_layer(p):
    s = p["gamma"] / jnp.sqrt(p["var"] + 1e-3)
    t = p["beta"] - p["mean"] * s
    return [s.reshape(1, -1), t.reshape(1, -1), p["W"], p["b"].reshape(1, -1)]


def _prep_ffn(layers):
    out = []
    for p in layers:
        out.extend(_prep_layer(p))
    return out


def _apply_ffn_refs(v, refs):
    for i in range(0, len(refs), 4):
        s, t, W, b = refs[i:i + 4]
        v = _gelu(jnp.dot(v * s[...] + t[...], W[...],
                          preferred_element_type=jnp.float32) + b[...])
    return v


def _full_spec(a):
    nd = a.ndim
    return pl.BlockSpec(a.shape, lambda i: (0,) * nd)


def _row_spec(cols):
    return pl.BlockSpec((RB, cols), lambda i: (i, 0))


def _stage_a(node, pre_flat, msg_flat):
    n_pre, n_msg = len(pre_flat), len(msg_flat)

    def body(node_ref, *refs):
        pre_refs = refs[:n_pre]
        msg_refs = refs[n_pre:n_pre + n_msg]
        x_out, y_out = refs[n_pre + n_msg:]
        v = _apply_ffn_refs(node_ref[...], pre_refs)
        x_out[...] = v
        y_out[...] = _apply_ffn_refs(v, msg_refs)

    flat = list(pre_flat) + list(msg_flat)
    return pl.pallas_call(
        body,
        grid=(GRID,),
        in_specs=[_row_spec(F)] + [_full_spec(a) for a in flat],
        out_specs=[_row_spec(H), _row_spec(H)],
        out_shape=[jax.ShapeDtypeStruct((N, H), jnp.float32)] * 2,
        compiler_params=pltpu.CompilerParams(
            dimension_semantics=("arbitrary",)),
    )(node, *flat)


def _stage_b(x, agg_a, agg_b, upd_split, msg_flat):
    n_upd, n_msg = len(upd_split), len(msg_flat)

    def body(x_ref, aa_ref, ab_ref, *refs):
        upd_refs = refs[:n_upd]
        msg_refs = refs[n_upd:n_upd + n_msg]
        x2_out, y_out = refs[n_upd + n_msg:]
        x_v = x_ref[...]
        agg = aa_ref[...] + ab_ref[...]
        sx, tx, sa, ta, Wx, Wa, b1 = upd_refs[:7]
        u = _gelu(jnp.dot(x_v * sx[...] + tx[...], Wx[...],
                          preferred_element_type=jnp.float32)
                  + jnp.dot(agg * sa[...] + ta[...], Wa[...],
                            preferred_element_type=jnp.float32) + b1[...])
        u = _apply_ffn_refs(u, upd_refs[7:])
        nrm = jnp.maximum(jnp.sum(u * u, axis=-1, keepdims=True), 1e-12)
        x2 = u * lax.rsqrt(nrm) + x_v
        x2_out[...] = x2
        y_out[...] = _apply_ffn_refs(x2, msg_refs)

    flat = list(upd_split) + list(msg_flat)
    return pl.pallas_call(
        body,
        grid=(GRID,),
        in_specs=[_row_spec(H)] * 3 + [_full_spec(a) for a in flat],
        out_specs=[_row_spec(H), _row_spec(H)],
        out_shape=[jax.ShapeDtypeStruct((N, H), jnp.float32)] * 2,
        compiler_params=pltpu.CompilerParams(
            dimension_semantics=("arbitrary",)),
    )(x, agg_a, agg_b, *flat)


def _stage_c(x, agg_a, agg_b, upd_split, post_flat, wl, bl, wc, bc):
    n_upd, n_post = len(upd_split), len(post_flat)

    def body(x_ref, aa_ref, ab_ref, *refs):
        upd_refs = refs[:n_upd]
        post_refs = refs[n_upd:n_upd + n_post]
        wl_ref, bl_ref, wc_ref, bc_ref, out_ref = refs[n_upd + n_post:]
        i = pl.program_id(0)
        x_v = x_ref[...]
        agg = aa_ref[...] + ab_ref[...]
        sx, tx, sa, ta, Wx, Wa, b1 = upd_refs[:7]
        u = _gelu(jnp.dot(x_v * sx[...] + tx[...], Wx[...],
                          preferred_element_type=jnp.float32)
                  + jnp.dot(agg * sa[...] + ta[...], Wa[...],
                            preferred_element_type=jnp.float32) + b1[...])
        u = _apply_ffn_refs(u, upd_refs[7:])
        nrm = jnp.maximum(jnp.sum(u * u, axis=-1, keepdims=True), 1e-12)
        x3 = u * lax.rsqrt(nrm) + x_v
        v = _apply_ffn_refs(x3, post_refs)
        logit = jnp.dot(v, wl_ref[...],
                        preferred_element_type=jnp.float32) + bl_ref[...]

        @pl.when(i == 0)
        def _():
            out_ref[...] = bc_ref[...]

        out_ref[...] += jnp.sum(logit * wc_ref[...]).reshape(1, 1)

    flat = list(upd_split) + list(post_flat)
    return pl.pallas_call(
        body,
        grid=(GRID,),
        in_specs=([_row_spec(H)] * 3 + [_full_spec(a) for a in flat]
                  + [_full_spec(wl), _full_spec(bl),
                     pl.BlockSpec((RB, 1), lambda i: (i, 0)),
                     _full_spec(bc)]),
        out_specs=pl.BlockSpec((1, 1), lambda i: (0, 0)),
        out_shape=jax.ShapeDtypeStruct((1, 1), jnp.float32),
        compiler_params=pltpu.CompilerParams(
            dimension_semantics=("arbitrary",)),
    )(x, agg_a, agg_b, *flat, wl, bl, wc, bc)


def _spmm(y, nbr1, dst1, bw1, zrows):
    mesh = plsc.VectorSubcoreMesh(core_axis_name="c", subcore_axis_name="s")

    @functools.partial(
        pl.kernel,
        out_type=jax.ShapeDtypeStruct((NC, N, H), jnp.float32),
        mesh=mesh,
        scratch_types=[
            pltpu.VMEM((WIN,), jnp.int32),
            pltpu.VMEM((KT, TPW), jnp.int32),
            pltpu.VMEM((WIN,), jnp.float32),
            pltpu.VMEM((WIN, H), jnp.float32),
            pltpu.VMEM_SHARED((N, H), jnp.float32),
            pltpu.SemaphoreType.DMA,
            pltpu.SemaphoreType.DMA,
        ],
        compiler_params=pltpu.CompilerParams(
            use_tc_tiling_on_sc=False, needs_layout_passes=False),
    )
    def k(y_hbm, nbr_hbm, dst_hbm, bw_hbm, z_hbm, out_hbm,
          gidx_v, sidx_v, bw_v, rows_v, agg_sh, sem, isem):
        c = lax.axis_index("c")
        s = lax.axis_index("s")
        wid = s * NC + c
        pltpu.sync_copy(z_hbm, agg_sh.at[pl.ds(s * NPT, NPT), :])
        plsc.subcore_barrier()

        def window(w, carry):
            eoff = wid * EPT + w * WIN
            idescs = [
                pltpu.async_copy(dst_hbm.at[pl.ds(eoff + j * TPW, TPW)],
                                 sidx_v.at[j], isem)
                for j in range(KT)
            ]
            pltpu.sync_copy(nbr_hbm.at[pl.ds(eoff, WIN)], gidx_v)
            pltpu.sync_copy(bw_hbm.at[pl.ds(eoff, WIN)], bw_v)
            descs = [
                pltpu.async_copy(y_hbm.at[gidx_v.at[pl.ds(j * TPW, TPW)]],
                                 rows_v.at[pl.ds(j * TPW, TPW), :], sem)
                for j in range(KT)
            ]
            for d in descs:
                d.wait()

            def mul(g, cc):
                for l in range(16):
                    e = g * 16 + l
                    splat = plsc.load_gather(
                        bw_v, [jnp.full((16,), e, jnp.int32)])
                    rows_v[e, :] = rows_v[e, :] * splat
                return cc

            lax.fori_loop(0, WIN // 16, mul, 0)

            for d in idescs:
                d.wait()
            sdescs = [
                pltpu.async_copy(rows_v.at[pl.ds(j * TPW, TPW), :],
                                 agg_sh.at[sidx_v.at[j]], sem, add=True)
                for j in range(KT)
            ]
            for d in sdescs:
                d.wait()
            return carry

        lax.fori_loop(0, NWIN, window, 0)
        plsc.subcore_barrier()
        pltpu.sync_copy(agg_sh.at[pl.ds(s * NPT, NPT), :],
                        out_hbm.at[c, pl.ds(s * NPT, NPT), :])

    return k(y, nbr1, dst1, bw1, zrows)


def _prep_upd_split(layers):
    p0 = layers[0]
    s = p0["gamma"] / jnp.sqrt(p0["var"] + 1e-3)
    t = p0["beta"] - p0["mean"] * s
    out = [s[:H].reshape(1, H), t[:H].reshape(1, H),
           s[H:].reshape(1, H), t[H:].reshape(1, H),
           p0["W"][:H], p0["W"][H:], p0["b"].reshape(1, -1)]
    for p in layers[1:]:
        out.extend(_prep_layer(p))
    return out


def kernel(node_info, branches, branch_info, params):
    node = node_info[0]
    nbr = branches[1].astype(jnp.int32)
    dst = branches[0].astype(jnp.int32)
    bw = branch_info[0, :, 0]

    pad = EPAD - E
    pidx = lax.iota(jnp.int32, pad) % N
    nbr1 = jnp.concatenate([nbr, pidx])
    dst1 = jnp.concatenate([dst, pidx])
    bw1 = jnp.concatenate([bw, jnp.zeros((pad,), jnp.float32)])
    zrows = jnp.zeros((NPT, H), jnp.float32)

    pre_flat = _prep_ffn(params["pre"])
    msg1_flat = _prep_ffn(params["conv1"]["msg"])
    msg2_flat = _prep_ffn(params["conv2"]["msg"])
    upd1_split = _prep_upd_split(params["conv1"]["upd"])
    upd2_split = _prep_upd_split(params["conv2"]["upd"])
    post_flat = _prep_ffn(params["post"])
    wl = params["logits"]["W"]
    bl = params["logits"]["b"].reshape(1, 1)
    wc = params["critic"]["W"]
    bc = params["critic"]["b"].reshape(1, 1)

    x, y1 = _stage_a(node, pre_flat, msg1_flat)
    agg1 = _spmm(y1, nbr1, dst1, bw1, zrows)
    x2, y2 = _stage_b(x, agg1[0], agg1[1], upd1_split, msg2_flat)
    agg2 = _spmm(y2, nbr1, dst1, bw1, zrows)
    return _stage_c(x2, agg2[0], agg2[1], upd2_split, post_flat, wl, bl, wc, bc)


# speedup vs baseline: 1.2017x; 1.1325x over previous
import functools

import jax
import jax.numpy as jnp
from jax import lax
from jax.experimental import pallas as pl
from jax.experimental.pallas import tpu as pltpu
from jax.experimental.pallas import tpu_sc as plsc

N = 50000
E = 800000
F = 128
H = 16

NC = 2
NS = 16
NW = NC * NS
TPW = 128
KT = 20
WIN = KT * TPW
NWIN = 10
EPT = WIN * NWIN
EPAD = EPT * NW
NPT = N // NS

RB = 5000
GRID = N // RB


def _gelu(v):
    return 0.5 * v * (1.0 + lax.erf(v * (2.0 ** -0.5)))


def _prep_layer(p):
    s = p["gamma"] / jnp.sqrt(p["var"] + 1e-3)
    t = p["beta"] - p["mean"] * s
    return [s.reshape(1, -1), t.reshape(1, -1), p["W"], p["b"].reshape(1, -1)]


def _prep_ffn(layers):
    out = []
    for p in layers:
        out.extend(_prep_layer(p))
    return out


def _apply_ffn_refs(v, refs):
    for i in range(0, len(refs), 4):
        s, t, W, b = refs[i:i + 4]
        v = _gelu(jnp.dot(v * s[...] + t[...], W[...],
                          preferred_element_type=jnp.float32) + b[...])
    return v


def _full_spec(a):
    nd = a.ndim
    return pl.BlockSpec(a.shape, lambda i: (0,) * nd)


def _row_spec(cols):
    return pl.BlockSpec((RB, cols), lambda i: (i, 0))


def _stage_a(node, pre_flat, msg_flat):
    n_pre, n_msg = len(pre_flat), len(msg_flat)

    def body(node_ref, *refs):
        pre_refs = refs[:n_pre]
        msg_refs = refs[n_pre:n_pre + n_msg]
        x_out, y_out = refs[n_pre + n_msg:]
        v = _apply_ffn_refs(node_ref[...], pre_refs)
        x_out[...] = v
        y_out[...] = _apply_ffn_refs(v, msg_refs)

    flat = list(pre_flat) + list(msg_flat)
    return pl.pallas_call(
        body,
        grid=(GRID,),
        in_specs=[_row_spec(F)] + [_full_spec(a) for a in flat],
        out_specs=[_row_spec(H), _row_spec(H)],
        out_shape=[jax.ShapeDtypeStruct((N, H), jnp.float32)] * 2,
        compiler_params=pltpu.CompilerParams(
            dimension_semantics=("arbitrary",)),
    )(node, *flat)


def _agg_spec():
    return pl.BlockSpec((NC, RB, H), lambda i: (0, i, 0))


def _stage_b(x, agg, upd_split, msg_flat):
    n_upd, n_msg = len(upd_split), len(msg_flat)

    def body(x_ref, agg_ref, *refs):
        upd_refs = refs[:n_upd]
        msg_refs = refs[n_upd:n_upd + n_msg]
        x2_out, y_out = refs[n_upd + n_msg:]
        x_v = x_ref[...]
        ag = agg_ref[...]
        agg = ag[0] + ag[1]
        sx, tx, sa, ta, Wx, Wa, b1 = upd_refs[:7]
        u = _gelu(jnp.dot(x_v * sx[...] + tx[...], Wx[...],
                          preferred_element_type=jnp.float32)
                  + jnp.dot(agg * sa[...] + ta[...], Wa[...],
                            preferred_element_type=jnp.float32) + b1[...])
        u = _apply_ffn_refs(u, upd_refs[7:])
        nrm = jnp.maximum(jnp.sum(u * u, axis=-1, keepdims=True), 1e-12)
        x2 = u * lax.rsqrt(nrm) + x_v
        x2_out[...] = x2
        y_out[...] = _apply_ffn_refs(x2, msg_refs)

    flat = list(upd_split) + list(msg_flat)
    return pl.pallas_call(
        body,
        grid=(GRID,),
        in_specs=[_row_spec(H), _agg_spec()] + [_full_spec(a) for a in flat],
        out_specs=[_row_spec(H), _row_spec(H)],
        out_shape=[jax.ShapeDtypeStruct((N, H), jnp.float32)] * 2,
        compiler_params=pltpu.CompilerParams(
            dimension_semantics=("arbitrary",)),
    )(x, agg, *flat)


def _stage_c(x, agg, upd_split, post_flat, wl, bl, wc, bc):
    n_upd, n_post = len(upd_split), len(post_flat)

    def body(x_ref, agg_ref, *refs):
        upd_refs = refs[:n_upd]
        post_refs = refs[n_upd:n_upd + n_post]
        wl_ref, bl_ref, wc_ref, bc_ref, out_ref = refs[n_upd + n_post:]
        i = pl.program_id(0)
        x_v = x_ref[...]
        ag = agg_ref[...]
        agg = ag[0] + ag[1]
        sx, tx, sa, ta, Wx, Wa, b1 = upd_refs[:7]
        u = _gelu(jnp.dot(x_v * sx[...] + tx[...], Wx[...],
                          preferred_element_type=jnp.float32)
                  + jnp.dot(agg * sa[...] + ta[...], Wa[...],
                            preferred_element_type=jnp.float32) + b1[...])
        u = _apply_ffn_refs(u, upd_refs[7:])
        nrm = jnp.maximum(jnp.sum(u * u, axis=-1, keepdims=True), 1e-12)
        x3 = u * lax.rsqrt(nrm) + x_v
        v = _apply_ffn_refs(x3, post_refs)
        logit = jnp.dot(v, wl_ref[...],
                        preferred_element_type=jnp.float32) + bl_ref[...]

        @pl.when(i == 0)
        def _():
            out_ref[...] = bc_ref[...]

        out_ref[...] += jnp.sum(logit * wc_ref[...]).reshape(1, 1)

    flat = list(upd_split) + list(post_flat)
    return pl.pallas_call(
        body,
        grid=(GRID,),
        in_specs=([_row_spec(H), _agg_spec()] + [_full_spec(a) for a in flat]
                  + [_full_spec(wl), _full_spec(bl),
                     pl.BlockSpec((RB, 1), lambda i: (i, 0)),
                     _full_spec(bc)]),
        out_specs=pl.BlockSpec((1, 1), lambda i: (0, 0)),
        out_shape=jax.ShapeDtypeStruct((1, 1), jnp.float32),
        compiler_params=pltpu.CompilerParams(
            dimension_semantics=("arbitrary",)),
    )(x, agg, *flat, wl, bl, wc, bc)


def _spmm(y, br, bw, zrows):
    mesh = plsc.VectorSubcoreMesh(core_axis_name="c", subcore_axis_name="s")

    @functools.partial(
        pl.kernel,
        out_type=jax.ShapeDtypeStruct((NC, N, H), jnp.float32),
        mesh=mesh,
        scratch_types=[
            pltpu.VMEM((WIN,), jnp.int32),
            pltpu.VMEM((KT, TPW), jnp.int32),
            pltpu.VMEM((WIN,), jnp.float32),
            pltpu.VMEM((WIN, H), jnp.float32),
            pltpu.VMEM_SHARED((N, H), jnp.float32),
            pltpu.SemaphoreType.DMA,
            pltpu.SemaphoreType.DMA,
        ],
        compiler_params=pltpu.CompilerParams(
            use_tc_tiling_on_sc=False, needs_layout_passes=False),
    )
    def k(y_hbm, br_hbm, bw_hbm, z_hbm, out_hbm,
          gidx_v, sidx_v, bw_v, rows_v, agg_sh, sem, isem):
        c = lax.axis_index("c")
        s = lax.axis_index("s")
        wid = s * NC + c
        pltpu.sync_copy(z_hbm, agg_sh.at[pl.ds(s * NPT, NPT), :])
        plsc.subcore_barrier()

        def process(base, kt):
            ne = kt * TPW
            idescs = [
                pltpu.async_copy(br_hbm.at[0, pl.ds(base + j * TPW, TPW)],
                                 sidx_v.at[j], isem)
                for j in range(kt)
            ]
            pltpu.sync_copy(br_hbm.at[1, pl.ds(base, ne)],
                            gidx_v.at[pl.ds(0, ne)])
            pltpu.sync_copy(bw_hbm.at[pl.ds(base, ne)],
                            bw_v.at[pl.ds(0, ne)])
            descs = [
                pltpu.async_copy(y_hbm.at[gidx_v.at[pl.ds(j * TPW, TPW)]],
                                 rows_v.at[pl.ds(j * TPW, TPW), :], sem)
                for j in range(kt)
            ]
            for d in descs:
                d.wait()

            def mul(g, cc):
                for l in range(16):
                    e = g * 16 + l
                    splat = plsc.load_gather(
                        bw_v, [jnp.full((16,), e, jnp.int32)])
                    rows_v[e, :] = rows_v[e, :] * splat
                return cc

            lax.fori_loop(0, ne // 16, mul, 0)

            for d in idescs:
                d.wait()
            sdescs = [
                pltpu.async_copy(rows_v.at[pl.ds(j * TPW, TPW), :],
                                 agg_sh.at[sidx_v.at[j]], sem, add=True)
                for j in range(kt)
            ]
            for d in sdescs:
                d.wait()

        def window(w, carry):
            process(wid * EPT + w * WIN, KT)
            return carry

        nwin = jnp.where(wid == NW - 1, 2, NWIN)
        lax.fori_loop(0, nwin, window, 0)

        @pl.when(wid == NW - 1)
        def _():
            process((NW - 1) * EPT + 2 * WIN, KT // 2)

        plsc.subcore_barrier()
        pltpu.sync_copy(agg_sh.at[pl.ds(s * NPT, NPT), :],
                        out_hbm.at[c, pl.ds(s * NPT, NPT), :])

    return k(y, br, bw, zrows)


def _prep_upd_split(layers):
    p0 = layers[0]
    s = p0["gamma"] / jnp.sqrt(p0["var"] + 1e-3)
    t = p0["beta"] - p0["mean"] * s
    out = [s[:H].reshape(1, H), t[:H].reshape(1, H),
           s[H:].reshape(1, H), t[H:].reshape(1, H),
           p0["W"][:H], p0["W"][H:], p0["b"].reshape(1, -1)]
    for p in layers[1:]:
        out.extend(_prep_layer(p))
    return out


def kernel(node_info, branches, branch_info, params):
    node = node_info[0]
    br = branches.astype(jnp.int32)
    bw = jnp.reshape(branch_info, (E,))
    zrows = jnp.zeros((NPT, H), jnp.float32)

    pre_flat = _prep_ffn(params["pre"])
    msg1_flat = _prep_ffn(params["conv1"]["msg"])
    msg2_flat = _prep_ffn(params["conv2"]["msg"])
    upd1_split = _prep_upd_split(params["conv1"]["upd"])
    upd2_split = _prep_upd_split(params["conv2"]["upd"])
    post_flat = _prep_ffn(params["post"])
    wl = params["logits"]["W"]
    bl = params["logits"]["b"].reshape(1, 1)
    wc = params["critic"]["W"]
    bc = params["critic"]["b"].reshape(1, 1)

    x, y1 = _stage_a(node, pre_flat, msg1_flat)
    agg1 = _spmm(y1, br, bw, zrows)
    x2, y2 = _stage_b(x, agg1, upd1_split, msg2_flat)
    agg2 = _spmm(y2, br, bw, zrows)
    return _stage_c(x2, agg2, upd2_split, post_flat, wl, bl, wc, bc)
